# baseline (device time: 19081 ns/iter reference)
import jax
import jax.numpy as jnp
from jax import lax
from jax.experimental import pallas as pl
from jax.experimental.pallas import tpu as pltpu

N_DEV = 16
HALO = 3


def kernel(x, k):
    b, s_per, c = x.shape
    n_taps = k.shape[0]

    def silu(a):
        return a * (1.0 / (1.0 + jnp.exp(-a)))

    def body(x_ref, k_ref, out_ref, halo_ref, send_sems, recv_sems):
        bi = pl.program_id(0)
        my = lax.axis_index("i")
        left = lax.rem(my + N_DEV - 1, N_DEV)
        right = lax.rem(my + 1, N_DEV)

        @pl.when(bi == 0)
        def _():
            barrier = pltpu.get_barrier_semaphore()
            for nbr in (left, right):
                pl.semaphore_signal(
                    barrier, inc=1,
                    device_id=(nbr,), device_id_type=pl.DeviceIdType.MESH,
                )
            pl.semaphore_wait(barrier, 2)

        rdma = pltpu.make_async_remote_copy(
            src_ref=x_ref.at[0, pl.ds(s_per - HALO, HALO), :],
            dst_ref=halo_ref.at[bi],
            send_sem=send_sems.at[bi],
            recv_sem=recv_sems.at[bi],
            device_id=(right,),
            device_id_type=pl.DeviceIdType.MESH,
        )
        rdma.start()

        kb = k_ref[...].astype(jnp.bfloat16)
        xb = x_ref[0].astype(jnp.bfloat16)
        p = jnp.concatenate(
            [jnp.zeros((HALO, c), jnp.bfloat16), xb], axis=0
        )
        acc = p[HALO:, :] * kb[n_taps - 1, :]
        for t in range(n_taps - 1):
            acc = acc + p[t:t + s_per, :] * kb[t, :]
        out_ref[0] = silu(acc)

        rdma.wait_recv()

        use_halo = (
            jnp.where(my == 0, 0.0, 1.0).astype(jnp.bfloat16)
            * halo_ref[bi].astype(jnp.bfloat16)
        )
        ph = jnp.concatenate([use_halo, xb[:HALO, :]], axis=0)
        accf = ph[HALO:, :] * kb[n_taps - 1, :]
        for t in range(n_taps - 1):
            accf = accf + ph[t:t + HALO, :] * kb[t, :]
        out_ref[0, :HALO] = silu(accf)

        rdma.wait_send()

    return pl.pallas_call(
        body,
        grid=(b,),
        out_shape=jax.ShapeDtypeStruct((b, s_per, c), jnp.bfloat16),
        in_specs=[
            pl.BlockSpec((1, s_per, c), lambda i: (i, 0, 0)),
            pl.BlockSpec((n_taps, c), lambda i: (0, 0)),
        ],
        out_specs=pl.BlockSpec((1, s_per, c), lambda i: (i, 0, 0)),
        scratch_shapes=[
            pltpu.VMEM((b, HALO, c), jnp.float32),
            pltpu.SemaphoreType.DMA((b,)),
            pltpu.SemaphoreType.DMA((b,)),
        ],
        compiler_params=pltpu.CompilerParams(collective_id=0),
    )(x, k)


# device time: 14664 ns/iter; 1.3012x vs baseline; 1.3012x over previous
import jax
import jax.numpy as jnp
from jax import lax
from jax.experimental import pallas as pl
from jax.experimental.pallas import tpu as pltpu

N_DEV = 16
HALO = 3


def kernel(x, k):
    b, s_per, c = x.shape
    n_taps = k.shape[0]

    def silu(a):
        return a * (1.0 / (1.0 + jnp.exp(-a)))

    def body(x_ref, k_ref, out_ref, halo_ref, send_sem, recv_sem):
        my = lax.axis_index("i")
        left = lax.rem(my + N_DEV - 1, N_DEV)
        right = lax.rem(my + 1, N_DEV)

        barrier = pltpu.get_barrier_semaphore()
        pl.semaphore_signal(
            barrier, inc=1,
            device_id=(left,), device_id_type=pl.DeviceIdType.MESH,
        )
        pl.semaphore_wait(barrier, 1)

        rdma = pltpu.make_async_remote_copy(
            src_ref=x_ref.at[:, pl.ds(s_per - HALO, HALO), :],
            dst_ref=halo_ref,
            send_sem=send_sem,
            recv_sem=recv_sem,
            device_id=(right,),
            device_id_type=pl.DeviceIdType.MESH,
        )
        rdma.start()

        kb = k_ref[...].astype(jnp.bfloat16)
        xb = x_ref[...].astype(jnp.bfloat16)
        p = jnp.concatenate(
            [jnp.zeros((b, HALO, c), jnp.bfloat16), xb], axis=1
        )
        acc = p[:, HALO:, :] * kb[n_taps - 1, :]
        for t in range(n_taps - 1):
            acc = acc + p[:, t:t + s_per, :] * kb[t, :]
        out_ref[...] = silu(acc)

        rdma.wait_recv()

        use_halo = (
            jnp.where(my == 0, 0.0, 1.0).astype(jnp.bfloat16)
            * halo_ref[...].astype(jnp.bfloat16)
        )
        ph = jnp.concatenate(
            [use_halo, xb[:, :HALO, :]], axis=1
        )
        accf = ph[:, HALO:, :] * kb[n_taps - 1, :]
        for t in range(n_taps - 1):
            accf = accf + ph[:, t:t + HALO, :] * kb[t, :]
        out_ref[:, :HALO, :] = silu(accf)

        rdma.wait_send()

    return pl.pallas_call(
        body,
        out_shape=jax.ShapeDtypeStruct((b, s_per, c), jnp.bfloat16),
        in_specs=[
            pl.BlockSpec(memory_space=pltpu.VMEM),
            pl.BlockSpec(memory_space=pltpu.VMEM),
        ],
        out_specs=pl.BlockSpec(memory_space=pltpu.VMEM),
        scratch_shapes=[
            pltpu.VMEM((b, HALO, c), jnp.float32),
            pltpu.SemaphoreType.DMA,
            pltpu.SemaphoreType.DMA,
        ],
        compiler_params=pltpu.CompilerParams(collective_id=0),
    )(x, k)


# device time: 12863 ns/iter; 1.4834x vs baseline; 1.1400x over previous
import jax
import jax.numpy as jnp
from jax import lax
from jax.experimental import pallas as pl
from jax.experimental.pallas import tpu as pltpu

N_DEV = 16
HALO = 3


def kernel(x, k):
    b, s_per, c = x.shape
    n_taps = k.shape[0]

    def silu(a):
        return (0.5 * a) * (1.0 + jnp.tanh(0.5 * a))

    def body(x_ref, k_ref, out_ref, halo_ref, send_sem, recv_sem):
        my = lax.axis_index("i")
        left = lax.rem(my + N_DEV - 1, N_DEV)
        right = lax.rem(my + 1, N_DEV)

        barrier = pltpu.get_barrier_semaphore()
        pl.semaphore_signal(
            barrier, inc=1,
            device_id=(left,), device_id_type=pl.DeviceIdType.MESH,
        )
        pl.semaphore_wait(barrier, 1)

        rdma = pltpu.make_async_remote_copy(
            src_ref=x_ref.at[:, pl.ds(s_per - HALO, HALO), :],
            dst_ref=halo_ref,
            send_sem=send_sem,
            recv_sem=recv_sem,
            device_id=(right,),
            device_id_type=pl.DeviceIdType.MESH,
        )
        rdma.start()

        kb = k_ref[...].astype(jnp.bfloat16)
        xb = x_ref[...].astype(jnp.bfloat16)
        acc = xb * kb[n_taps - 1, :]
        for t in range(n_taps - 1):
            acc = acc + pltpu.roll(xb, n_taps - 1 - t, 1) * kb[t, :]
        out_ref[...] = silu(acc)

        rdma.wait_recv()

        use_halo = (
            jnp.where(my == 0, 0.0, 1.0).astype(jnp.bfloat16)
            * halo_ref[...].astype(jnp.bfloat16)
        )
        ph = jnp.concatenate(
            [use_halo, xb[:, :HALO, :]], axis=1
        )
        accf = ph[:, HALO:, :] * kb[n_taps - 1, :]
        for t in range(n_taps - 1):
            accf = accf + ph[:, t:t + HALO, :] * kb[t, :]
        out_ref[:, :HALO, :] = silu(accf)

        rdma.wait_send()

    return pl.pallas_call(
        body,
        out_shape=jax.ShapeDtypeStruct((b, s_per, c), jnp.bfloat16),
        in_specs=[
            pl.BlockSpec(memory_space=pltpu.VMEM),
            pl.BlockSpec(memory_space=pltpu.VMEM),
        ],
        out_specs=pl.BlockSpec(memory_space=pltpu.VMEM),
        scratch_shapes=[
            pltpu.VMEM((b, HALO, c), jnp.float32),
            pltpu.SemaphoreType.DMA,
            pltpu.SemaphoreType.DMA,
        ],
        compiler_params=pltpu.CompilerParams(collective_id=0),
    )(x, k)
